# baseline (speedup 1.0000x reference)
import functools

import jax
import jax.numpy as jnp
from jax import lax
from jax.experimental import pallas as pl
from jax.experimental.pallas import tpu as pltpu
from jax.experimental.pallas import tpu_sc as plsc

N_NODES = 10000
N_EDGES = 320000
N_AA = 500
N_AA_EDGES = 16000

NT1 = 10240
E1P = 327680
NTAB5 = 512
E5P = 16384

F32 = jnp.float32


def _node_proj_kernel(x_ref, wl_ref, ws_ref, wd_ref, wa_ref,
                      vbs_ref, bd_ref, ms_ref, md_ref):
    i = pl.program_id(0)
    xb = x_ref[...]
    v = jnp.dot(xb, wl_ref[...], preferred_element_type=F32)
    asrc = jnp.dot(xb, ws_ref[...], preferred_element_type=F32)
    adst = jnp.dot(xb, wd_ref[...], preferred_element_type=F32)
    bs_parts = []
    bd_parts = []
    for c in range(3):
        wa = wa_ref[c]
        bs_parts.append(jnp.dot(asrc[:, 128 * c:128 * (c + 1)], wa,
                                preferred_element_type=F32))
        bd_parts.append(jnp.dot(adst[:, 128 * c:128 * (c + 1)], wa,
                                preferred_element_type=F32))
    bs = jnp.concatenate(bs_parts, axis=1)
    bd = jnp.concatenate(bd_parts, axis=1)
    for j in range(6):
        sl = slice(64 * j, 64 * (j + 1))
        vbs_ref[j] = jnp.concatenate([v[:, sl], bs[:, sl]], axis=1)
        bd_ref[j] = bd[:, sl]

    @pl.when(i == 0)
    def _():
        ms_ref[0, 0] = 0.0
        md_ref[0, 0] = 0.0

    ms_ref[0, 0] = jnp.maximum(ms_ref[0, 0], jnp.abs(bs).max())
    md_ref[0, 0] = jnp.maximum(md_ref[0, 0], jnp.abs(bd).max())


def _node_proj(x_p, wl, ws, wd, wa3):
    R = 1024
    return pl.pallas_call(
        _node_proj_kernel,
        grid=(NT1 // R,),
        in_specs=[
            pl.BlockSpec((R, 128), lambda i: (i, 0)),
            pl.BlockSpec((128, 384), lambda i: (0, 0)),
            pl.BlockSpec((128, 384), lambda i: (0, 0)),
            pl.BlockSpec((128, 384), lambda i: (0, 0)),
            pl.BlockSpec((3, 128, 128), lambda i: (0, 0, 0)),
        ],
        out_specs=[
            pl.BlockSpec((6, R, 128), lambda i: (0, i, 0)),
            pl.BlockSpec((6, R, 64), lambda i: (0, i, 0)),
            pl.BlockSpec(memory_space=pltpu.SMEM, block_shape=(1, 1),
                         index_map=lambda i: (0, 0)),
            pl.BlockSpec(memory_space=pltpu.SMEM, block_shape=(1, 1),
                         index_map=lambda i: (0, 0)),
        ],
        out_shape=[
            jax.ShapeDtypeStruct((6, NT1, 128), F32),
            jax.ShapeDtypeStruct((6, NT1, 64), F32),
            jax.ShapeDtypeStruct((1, 1), F32),
            jax.ShapeDtypeStruct((1, 1), F32),
        ],
    )(x_p, wl, ws, wd, wa3)


def _edge_mm_kernel(rel_ref, wp_ref, bp_ref, wa_ref, ba_ref, td_ref, mt_ref):
    i = pl.program_id(0)
    delta = jnp.maximum(
        jnp.dot(rel_ref[...], wp_ref[...], preferred_element_type=F32)
        + bp_ref[...], 0.0)
    t_parts = []
    for c in range(3):
        t_parts.append(jnp.dot(delta[:, 128 * c:128 * (c + 1)], wa_ref[c],
                               preferred_element_type=F32))
    t0 = jnp.concatenate(t_parts, axis=1) + ba_ref[...]
    for j in range(6):
        sl = slice(64 * j, 64 * (j + 1))
        td_ref[j] = jnp.concatenate([t0[:, sl], delta[:, sl]], axis=1)

    @pl.when(i == 0)
    def _():
        mt_ref[0, 0] = 0.0

    mt_ref[0, 0] = jnp.maximum(mt_ref[0, 0], t0.max())


def _edge_mm(rel, wp, bp, wa3, ba):
    EB = 1024
    return pl.pallas_call(
        _edge_mm_kernel,
        grid=(E1P // EB,),
        in_specs=[
            pl.BlockSpec((EB, 16), lambda i: (i, 0)),
            pl.BlockSpec((16, 384), lambda i: (0, 0)),
            pl.BlockSpec((1, 384), lambda i: (0, 0)),
            pl.BlockSpec((3, 128, 128), lambda i: (0, 0, 0)),
            pl.BlockSpec((1, 384), lambda i: (0, 0)),
        ],
        out_specs=[
            pl.BlockSpec((6, EB, 128), lambda i: (0, i, 0)),
            pl.BlockSpec(memory_space=pltpu.SMEM, block_shape=(1, 1),
                         index_map=lambda i: (0, 0)),
        ],
        out_shape=[
            jax.ShapeDtypeStruct((6, E1P, 128), F32),
            jax.ShapeDtypeStruct((1, 1), F32),
        ],
    )(rel, wp, bp, wa3, ba)


def _neck_kernel(h1_ref, h2_ref, h3_ref, w_ref, b_ref, g_ref, be_ref, o_ref):
    h = jnp.concatenate([h1_ref[0], h1_ref[1], h2_ref[0], h2_ref[1],
                         h3_ref[0], h3_ref[1]], axis=1)
    z = jnp.dot(h, w_ref[...], preferred_element_type=F32) + b_ref[...]
    mu = jnp.mean(z, axis=-1, keepdims=True)
    var = jnp.mean((z - mu) * (z - mu), axis=-1, keepdims=True)
    zn = (z - mu) * lax.rsqrt(var + 1e-5) * g_ref[...] + be_ref[...]
    o_ref[...] = jax.nn.gelu(zn)


def _neck(h1, h2, h3, w, b, g, be):
    R = 1000
    return pl.pallas_call(
        _neck_kernel,
        grid=(N_NODES // R,),
        in_specs=[
            pl.BlockSpec((2, R, 64), lambda i: (0, i, 0)),
            pl.BlockSpec((2, R, 64), lambda i: (0, i, 0)),
            pl.BlockSpec((2, R, 64), lambda i: (0, i, 0)),
            pl.BlockSpec((384, 1280), lambda i: (0, 0)),
            pl.BlockSpec((1, 1280), lambda i: (0, 0)),
            pl.BlockSpec((1, 1280), lambda i: (0, 0)),
            pl.BlockSpec((1, 1280), lambda i: (0, 0)),
        ],
        out_specs=pl.BlockSpec((R, 1280), lambda i: (i, 0)),
        out_shape=jax.ShapeDtypeStruct((N_NODES, 1280), F32),
    )(h1, h2, h3, w, b, g, be)


def _pool_kernel(h_ref, o_ref):
    o_ref[...] = jnp.mean(h_ref[...], axis=1)


def _pool(h_r, cols):
    CB = min(cols, 256)
    return pl.pallas_call(
        _pool_kernel,
        grid=(cols // CB,),
        in_specs=[pl.BlockSpec((N_AA, 20, CB), lambda i: (0, 0, i))],
        out_specs=pl.BlockSpec((N_AA, CB), lambda i: (0, i)),
        out_shape=jax.ShapeDtypeStruct((N_AA, cols), F32),
    )(h_r)


def _aapos_kernel(p_ref, o_ref):
    ap = jnp.mean(p_ref[...], axis=1)
    c = ap - jnp.mean(ap, axis=0, keepdims=True)
    nrm = jnp.sqrt(jnp.sum(c * c, axis=1, keepdims=True))
    an = c / (nrm + 1e-8)
    o_ref[...] = jnp.concatenate([ap, an], axis=1)


def _aapos(pos_r):
    return pl.pallas_call(
        _aapos_kernel,
        grid=(1,),
        in_specs=[pl.BlockSpec((N_AA, 20, 8), lambda i: (0, 0, 0))],
        out_specs=pl.BlockSpec((N_AA, 16), lambda i: (0, 0)),
        out_shape=jax.ShapeDtypeStruct((N_AA, 16), F32),
    )(pos_r)


def _mm_kernel(x_ref, w_ref, o_ref):
    o_ref[...] = jnp.dot(x_ref[...], w_ref[...], preferred_element_type=F32)


def _mm(x, w, nb):
    M, K = x.shape
    N = w.shape[1]
    return pl.pallas_call(
        _mm_kernel,
        grid=(N // nb,),
        in_specs=[
            pl.BlockSpec((M, K), lambda i: (0, 0)),
            pl.BlockSpec((K, nb), lambda i: (0, i)),
        ],
        out_specs=pl.BlockSpec((M, nb), lambda i: (0, i)),
        out_shape=jax.ShapeDtypeStruct((M, N), F32),
    )(x, w)


def _node5_kernel(al_ref, as_ref, ad_ref, wa_ref, vbs_ref, bd_ref,
                  ms_ref, md_ref):
    n = pl.program_id(0)
    wa = wa_ref[...]
    bs = jnp.dot(as_ref[...], wa, preferred_element_type=F32)
    bd = jnp.dot(ad_ref[...], wa, preferred_element_type=F32)
    vbs_ref[0] = jnp.concatenate([al_ref[...], bs], axis=1)
    bd_ref[0] = bd

    @pl.when(n == 0)
    def _():
        ms_ref[0, 0] = 0.0
        md_ref[0, 0] = 0.0

    ms_ref[0, 0] = jnp.maximum(ms_ref[0, 0], jnp.abs(bs).max())
    md_ref[0, 0] = jnp.maximum(md_ref[0, 0], jnp.abs(bd).max())


def _node5(a5, wa5):
    return pl.pallas_call(
        _node5_kernel,
        grid=(10,),
        in_specs=[
            pl.BlockSpec((NTAB5, 128), lambda n: (0, n)),
            pl.BlockSpec((NTAB5, 1280), lambda n: (0, 1)),
            pl.BlockSpec((NTAB5, 1280), lambda n: (0, 2)),
            pl.BlockSpec((1280, 128), lambda n: (0, n)),
        ],
        out_specs=[
            pl.BlockSpec((1, NTAB5, 256), lambda n: (n // 5, 0, n % 5)),
            pl.BlockSpec((1, NTAB5, 128), lambda n: (n // 5, 0, n % 5)),
            pl.BlockSpec(memory_space=pltpu.SMEM, block_shape=(1, 1),
                         index_map=lambda n: (0, 0)),
            pl.BlockSpec(memory_space=pltpu.SMEM, block_shape=(1, 1),
                         index_map=lambda n: (0, 0)),
        ],
        out_shape=[
            jax.ShapeDtypeStruct((2, NTAB5, 1280), F32),
            jax.ShapeDtypeStruct((2, NTAB5, 640), F32),
            jax.ShapeDtypeStruct((1, 1), F32),
            jax.ShapeDtypeStruct((1, 1), F32),
        ],
    )(a5, a5, a5, wa5)


def _edge5_kernel(rel_ref, wp_ref, bp_ref, wa_ref, ba_ref, t_ref, d_ref,
                  mt_ref):
    e = pl.program_id(0)
    n = pl.program_id(1)
    delta = jnp.maximum(
        jnp.dot(rel_ref[...], wp_ref[...], preferred_element_type=F32)
        + bp_ref[...], 0.0)
    t0 = jnp.dot(delta, wa_ref[...], preferred_element_type=F32) + ba_ref[...]
    t_ref[0] = t0

    @pl.when(n % 5 == 0)
    def _():
        d_ref[0] = jnp.where(n < 5, delta[:, :640], delta[:, 640:])

    @pl.when(jnp.logical_and(e == 0, n == 0))
    def _():
        mt_ref[0, 0] = 0.0

    mt_ref[0, 0] = jnp.maximum(mt_ref[0, 0], t0.max())


def _edge5(rel5, wp, bp, wa, ba):
    EB = 800
    return pl.pallas_call(
        _edge5_kernel,
        grid=(E5P // EB, 10),
        in_specs=[
            pl.BlockSpec((EB, 16), lambda e, n: (e, 0)),
            pl.BlockSpec((16, 1280), lambda e, n: (0, 0)),
            pl.BlockSpec((1, 1280), lambda e, n: (0, 0)),
            pl.BlockSpec((1280, 128), lambda e, n: (0, n)),
            pl.BlockSpec((1, 128), lambda e, n: (0, n)),
        ],
        out_specs=[
            pl.BlockSpec((1, EB, 128), lambda e, n: (n // 5, e, n % 5)),
            pl.BlockSpec((1, EB, 640), lambda e, n: (n // 5, e, 0)),
            pl.BlockSpec(memory_space=pltpu.SMEM, block_shape=(1, 1),
                         index_map=lambda e, n: (0, 0)),
        ],
        out_shape=[
            jax.ShapeDtypeStruct((2, E5P, 640), F32),
            jax.ShapeDtypeStruct((2, E5P, 640), F32),
            jax.ShapeDtypeStruct((1, 1), F32),
        ],
    )(rel5, wp, bp, wa, ba)


def _towers_kernel(h_ref, esm_ref, w1_ref, b1_ref, g1_ref, e1_ref,
                   w2_ref, b2_ref, g2_ref, e2_ref,
                   w3_ref, b3_ref, g3_ref, e3_ref,
                   w4_ref, b4_ref, wo_ref, bo_ref, mk_ref, o_ref):
    def ln_gelu(z, nch, g, be):
        colmask = (lax.broadcasted_iota(jnp.int32, z.shape, 1) < nch)
        zm = jnp.where(colmask, z, 0.0)
        mu = jnp.sum(zm, axis=-1, keepdims=True) / nch
        dv = jnp.where(colmask, z - mu, 0.0)
        var = jnp.sum(dv * dv, axis=-1, keepdims=True) / nch
        zn = (z - mu) * lax.rsqrt(var + 1e-5) * g + be
        return jax.nn.gelu(jnp.where(colmask, zn, 0.0))

    h2 = jnp.concatenate([h_ref[0], h_ref[1]], axis=1) + esm_ref[...]
    t1 = ln_gelu(jnp.dot(h2, w1_ref[...], preferred_element_type=F32)
                 + b1_ref[...], 150, g1_ref[...], e1_ref[...])
    t2 = ln_gelu(jnp.dot(t1, w2_ref[...], preferred_element_type=F32)
                 + b2_ref[...], 120, g2_ref[...], e2_ref[...])
    t3 = ln_gelu(jnp.dot(t2, w3_ref[...], preferred_element_type=F32)
                 + b3_ref[...], 45, g3_ref[...], e3_ref[...])
    t4 = jnp.dot(t3, w4_ref[...], preferred_element_type=F32) + b4_ref[...]
    z = (jnp.dot(esm_ref[...], wo_ref[...], preferred_element_type=F32)
         + bo_ref[...])
    col = 1.0 / (1.0 + jnp.exp(-z))
    mp = jnp.max(mk_ref[...], axis=1, keepdims=True)
    o_ref[...] = (t4 + col) * mp


def _towers(h5, esm, tw, mask20):
    return pl.pallas_call(
        _towers_kernel,
        grid=(1,),
        in_specs=[
            pl.BlockSpec((2, NTAB5, 640), lambda i: (0, 0, 0)),
            pl.BlockSpec((NTAB5, 1280), lambda i: (0, 0)),
            pl.BlockSpec((1280, 256), lambda i: (0, 0)),
            pl.BlockSpec((1, 256), lambda i: (0, 0)),
            pl.BlockSpec((1, 256), lambda i: (0, 0)),
            pl.BlockSpec((1, 256), lambda i: (0, 0)),
            pl.BlockSpec((256, 128), lambda i: (0, 0)),
            pl.BlockSpec((1, 128), lambda i: (0, 0)),
            pl.BlockSpec((1, 128), lambda i: (0, 0)),
            pl.BlockSpec((1, 128), lambda i: (0, 0)),
            pl.BlockSpec((128, 128), lambda i: (0, 0)),
            pl.BlockSpec((1, 128), lambda i: (0, 0)),
            pl.BlockSpec((1, 128), lambda i: (0, 0)),
            pl.BlockSpec((1, 128), lambda i: (0, 0)),
            pl.BlockSpec((128, 128), lambda i: (0, 0)),
            pl.BlockSpec((1, 128), lambda i: (0, 0)),
            pl.BlockSpec((1280, 128), lambda i: (0, 0)),
            pl.BlockSpec((1, 128), lambda i: (0, 0)),
            pl.BlockSpec((NTAB5, 32), lambda i: (0, 0)),
        ],
        out_specs=pl.BlockSpec((NTAB5, 128), lambda i: (0, 0)),
        out_shape=jax.ShapeDtypeStruct((NTAB5, 128), F32),
    )(h5, esm, *tw, mask20)


def _sc_mesh():
    return plsc.VectorSubcoreMesh(core_axis_name="c", subcore_axis_name="s")


_SC_PARAMS = pltpu.CompilerParams(use_tc_tiling_on_sc=False)


def _rel_gather(ptab, srci, dsti, n_edges, block):
    ew = n_edges // 32
    iters = ew // block

    @functools.partial(
        pl.kernel,
        out_type=jax.ShapeDtypeStruct((n_edges, 16), F32),
        mesh=_sc_mesh(),
        compiler_params=_SC_PARAMS,
        scratch_types=[
            pltpu.VMEM((block,), jnp.int32),
            pltpu.VMEM((block,), jnp.int32),
            pltpu.VMEM((block, 16), F32),
            pltpu.VMEM((block, 16), F32),
            pltpu.VMEM((block, 16), F32),
            pltpu.SemaphoreType.DMA,
        ],
    )
    def k(p_hbm, s_hbm, d_hbm, rel_hbm, si_v, di_v, ps_v, pd_v, rl_v, sem):
        c = lax.axis_index("c")
        s = lax.axis_index("s")
        wid = s * 2 + c
        base = wid * ew

        def body(j, carry):
            b0 = base + j * block
            pltpu.sync_copy(s_hbm.at[pl.ds(b0, block)], si_v)
            pltpu.sync_copy(d_hbm.at[pl.ds(b0, block)], di_v)
            cp1 = pltpu.async_copy(p_hbm.at[si_v], ps_v, sem)
            cp2 = pltpu.async_copy(p_hbm.at[di_v], pd_v, sem)
            cp1.wait()
            cp2.wait()

            @plsc.parallel_loop(0, block, 1, unroll=8)
            def row(r):
                rl_v[r] = pd_v[r] - ps_v[r]
            pltpu.sync_copy(rl_v, rel_hbm.at[pl.ds(b0, block)])
            return carry

        lax.fori_loop(0, iters, body, 0)

    return k(ptab, srci, dsti)


def _agg123(td, vbs, bd, srci, dsti, cvec, zeros):
    n_edges, n_tab, n_acc, dh, B = E1P, NT1, NT1, 64, 64
    et = n_edges // 16
    iters = et // B
    n2 = iters // 2
    rows_pt = n_acc // 16
    rchunk = 32
    riters = rows_pt // rchunk

    @functools.partial(
        pl.kernel,
        out_type=jax.ShapeDtypeStruct((2 * n_acc, dh), F32),
        mesh=_sc_mesh(),
        compiler_params=_SC_PARAMS,
        scratch_types=[
            [pltpu.VMEM((B,), jnp.int32)] * 2,
            [pltpu.VMEM((B,), jnp.int32)] * 2,
            [pltpu.VMEM((B,), jnp.int32)] * 2,
            [pltpu.VMEM((B,), jnp.int32)] * 2,
            [pltpu.VMEM((B, 2 * dh), F32)] * 2,
            [pltpu.VMEM((B, 2 * dh), F32)] * 2,
            [pltpu.VMEM((B, dh), F32)] * 2,
            pltpu.VMEM((16,), F32),
            pltpu.VMEM((rchunk, 2 * dh), F32),
            pltpu.VMEM((rchunk, dh), F32),
            pltpu.VMEM_SHARED((n_acc, 2 * dh), F32),
            [pltpu.SemaphoreType.DMA] * 2,
        ],
    )
    def k(td_hbm, vbs_hbm, bd_hbm, s_hbm, dd_hbm, c_hbm, z_hbm, o_hbm,
          si_v, di_v, gs_v, gd_v, td_v, vbs_v, bd_v, c_v, rd_v, ob_v,
          acc, sems):
        c = lax.axis_index("c")
        s = lax.axis_index("s")
        pltpu.sync_copy(z_hbm.at[pl.ds(s * rows_pt, rows_pt)],
                        acc.at[pl.ds(s * rows_pt, rows_pt)])
        pltpu.sync_copy(c_hbm, c_v)
        plsc.subcore_barrier()
        cval = c_v[...]
        ebase = s * et
        toff = c * n_edges
        noff = c * n_tab

        def issue(sl, b0):
            pltpu.sync_copy(s_hbm.at[pl.ds(b0, B)], si_v[sl])
            pltpu.sync_copy(dd_hbm.at[pl.ds(b0, B)], di_v[sl])

            @plsc.parallel_loop(0, B // 16, 1, unroll=4)
            def oset(r):
                gs_v[sl][pl.ds(r * 16, 16)] = (
                    si_v[sl][pl.ds(r * 16, 16)] + noff)
                gd_v[sl][pl.ds(r * 16, 16)] = (
                    di_v[sl][pl.ds(r * 16, 16)] + noff)
            pltpu.async_copy(td_hbm.at[pl.ds(toff + b0, B)], td_v[sl],
                             sems[sl])
            pltpu.async_copy(vbs_hbm.at[gs_v[sl]], vbs_v[sl], sems[sl])
            pltpu.async_copy(bd_hbm.at[gd_v[sl]], bd_v[sl], sems[sl])

        def wait(sl, b0):
            pltpu.make_async_copy(td_hbm.at[pl.ds(toff + b0, B)], td_v[sl],
                                  sems[sl]).wait()
            pltpu.make_async_copy(vbs_hbm.at[gs_v[sl]], vbs_v[sl],
                                  sems[sl]).wait()
            pltpu.make_async_copy(bd_hbm.at[gd_v[sl]], bd_v[sl],
                                  sems[sl]).wait()

        def compute_scatter(sl):
            tdb = td_v[sl]
            vbsb = vbs_v[sl]
            bdb = bd_v[sl]

            @plsc.parallel_loop(0, B, 1, unroll=16)
            def rows(r):
                for kk in range(4):
                    a = pl.ds(16 * kk, 16)
                    b = pl.ds(dh + 16 * kk, 16)
                    t0 = tdb[r, a]
                    d = tdb[r, b]
                    al = jnp.maximum(t0 + bdb[r, a] - vbsb[r, b], 0.0)
                    e = jnp.exp(al - cval)
                    tdb[r, a] = e
                    tdb[r, b] = e * (vbsb[r, a] + d)

            pltpu.sync_copy(tdb, acc.at[di_v[sl]], add=True)

        issue(0, ebase)

        def body(j2, carry):
            jA = ebase + (2 * j2) * B
            jB = jA + B
            issue(1, jB)
            wait(0, jA)
            compute_scatter(0)

            @pl.when(j2 < n2 - 1)
            def _():
                issue(0, jB + B)

            wait(1, jB)
            compute_scatter(1)
            return carry

        lax.fori_loop(0, n2, body, 0)
        plsc.subcore_barrier()

        def rbody(j, carry):
            r0 = s * rows_pt + j * rchunk
            pltpu.sync_copy(acc.at[pl.ds(r0, rchunk)], rd_v)

            @plsc.parallel_loop(0, rchunk, 1, unroll=8)
            def rrow(r):
                for kk in range(4):
                    ob_v[r, pl.ds(16 * kk, 16)] = (
                        rd_v[r, pl.ds(dh + 16 * kk, 16)]
                        / (rd_v[r, pl.ds(16 * kk, 16)] + 1e-30))
            pltpu.sync_copy(ob_v, o_hbm.at[pl.ds(c * n_acc + r0, rchunk)])
            return carry

        lax.fori_loop(0, riters, rbody, 0)

    return k(td, vbs, bd, srci, dsti, cvec, zeros)


def _agg5(t5, d5, vbs5, bd5, srci, dsti, cvec, zeros):
    n_edges, n_tab, n_acc, dh, B = E5P, NTAB5, NTAB5, 640, 16
    et = n_edges // 16
    iters = et // B
    rows_pt = n_acc // 16
    rchunk = 8
    riters = rows_pt // rchunk

    @functools.partial(
        pl.kernel,
        out_type=jax.ShapeDtypeStruct((2 * n_acc, dh), F32),
        mesh=_sc_mesh(),
        compiler_params=_SC_PARAMS,
        scratch_types=[
            pltpu.VMEM((B,), jnp.int32),
            pltpu.VMEM((B,), jnp.int32),
            pltpu.VMEM((B,), jnp.int32),
            pltpu.VMEM((B,), jnp.int32),
            pltpu.VMEM((B, dh), F32),
            pltpu.VMEM((B, dh), F32),
            pltpu.VMEM((B, 2 * dh), F32),
            pltpu.VMEM((B, dh), F32),
            pltpu.VMEM((16,), F32),
            pltpu.VMEM((rchunk, dh), F32),
            pltpu.VMEM((rchunk, dh), F32),
            pltpu.VMEM_SHARED((n_acc, dh), F32),
            pltpu.VMEM_SHARED((n_acc, dh), F32),
            pltpu.SemaphoreType.DMA,
        ],
    )
    def k(t_hbm, d_hbm, vbs_hbm, bd_hbm, s_hbm, dd_hbm, c_hbm, z_hbm, o_hbm,
          si_v, di_v, gs_v, gd_v, t_v, d_v, vbs_v, bd_v, c_v, rn_v, rdn_v,
          accn, accd, sem):
        c = lax.axis_index("c")
        s = lax.axis_index("s")
        pltpu.sync_copy(z_hbm.at[pl.ds(s * rows_pt, rows_pt)],
                        accn.at[pl.ds(s * rows_pt, rows_pt)])
        pltpu.sync_copy(z_hbm.at[pl.ds(s * rows_pt, rows_pt)],
                        accd.at[pl.ds(s * rows_pt, rows_pt)])
        pltpu.sync_copy(c_hbm, c_v)
        plsc.subcore_barrier()
        cval = c_v[...]
        ebase = s * et
        toff = c * n_edges
        noff = c * n_tab

        def body(j, carry):
            b0 = ebase + j * B
            pltpu.sync_copy(s_hbm.at[pl.ds(b0, B)], si_v)
            pltpu.sync_copy(dd_hbm.at[pl.ds(b0, B)], di_v)
            gs_v[...] = si_v[...] + noff
            gd_v[...] = di_v[...] + noff
            cp1 = pltpu.async_copy(t_hbm.at[pl.ds(toff + b0, B)], t_v, sem)
            cp2 = pltpu.async_copy(d_hbm.at[pl.ds(toff + b0, B)], d_v, sem)
            cp3 = pltpu.async_copy(vbs_hbm.at[gs_v], vbs_v, sem)
            cp4 = pltpu.async_copy(bd_hbm.at[gd_v], bd_v, sem)
            cp1.wait()
            cp2.wait()
            cp3.wait()
            cp4.wait()

            @plsc.parallel_loop(0, B, 1, unroll=4)
            def rows(r):
                for kk in range(40):
                    vcol = 256 * (kk // 8) + 16 * (kk % 8)
                    a = pl.ds(16 * kk, 16)
                    t0 = t_v[r, a]
                    d = d_v[r, a]
                    al = jnp.maximum(
                        t0 + bd_v[r, a] - vbs_v[r, pl.ds(vcol + 128, 16)],
                        0.0)
                    e = jnp.exp(al - cval)
                    t_v[r, a] = e
                    d_v[r, a] = e * (vbs_v[r, pl.ds(vcol, 16)] + d)
            pltpu.sync_copy(d_v, accn.at[di_v], add=True)
            pltpu.sync_copy(t_v, accd.at[di_v], add=True)
            return carry

        lax.fori_loop(0, iters, body, 0)
        plsc.subcore_barrier()

        def rbody(j, carry):
            r0 = s * rows_pt + j * rchunk
            pltpu.sync_copy(accn.at[pl.ds(r0, rchunk)], rn_v)
            pltpu.sync_copy(accd.at[pl.ds(r0, rchunk)], rdn_v)

            @plsc.parallel_loop(0, rchunk, 1, unroll=2)
            def rrow(r):
                for kk in range(40):
                    a = pl.ds(16 * kk, 16)
                    rn_v[r, a] = rn_v[r, a] / (rdn_v[r, a] + 1e-30)
            pltpu.sync_copy(rn_v, o_hbm.at[pl.ds(c * n_acc + r0, rchunk)])
            return carry

        lax.fori_loop(0, riters, rbody, 0)

    return k(t5, d5, vbs5, bd5, srci, dsti, cvec, zeros)


def _pad_cols(a, n):
    return jnp.pad(a, ((0, 0), (0, n - a.shape[1])))


def _pad_rows(a, n):
    return jnp.pad(a, ((0, n - a.shape[0]), (0, 0)))


def kernel(x, pos, normal, mask, esm_list, edge_index, aa_edge_index,
           pool_batch, params):
    p1, p2, p3, p5 = (params["conv1"], params["conv2"], params["conv3"],
                      params["conv5"])
    epad = jnp.full((E1P - N_EDGES,), N_NODES, jnp.int32)
    srci = jnp.concatenate([edge_index[0], epad])
    dsti = jnp.concatenate([edge_index[1], epad])

    x_p = _pad_rows(_pad_cols(x, 128), NT1)
    wl = _pad_rows(jnp.concatenate([p1["W_lin"], p2["W_lin"], p3["W_lin"]],
                                   axis=1), 128)
    ws = _pad_rows(jnp.concatenate([p1["W_src"], p2["W_src"], p3["W_src"]],
                                   axis=1), 128)
    wd = _pad_rows(jnp.concatenate([p1["W_dst"], p2["W_dst"], p3["W_dst"]],
                                   axis=1), 128)
    wa3 = jnp.stack([p1["W_attn"], p2["W_attn"], p3["W_attn"]])
    vbs6, bd6, msrc, mdst = _node_proj(x_p, wl, ws, wd, wa3)

    ptab = _pad_rows(jnp.concatenate(
        [_pad_cols(pos, 8), _pad_cols(normal, 8)], axis=1), NT1)
    rel = _rel_gather(ptab, srci, dsti, E1P, 128)

    wp = jnp.zeros((16, 384), F32)
    wp_all = jnp.concatenate([p1["W_pos"], p2["W_pos"], p3["W_pos"]], axis=1)
    wp = wp.at[0:3].set(wp_all[0:3]).at[8:11].set(wp_all[3:6])
    bp = jnp.concatenate([p1["b_pos"], p2["b_pos"], p3["b_pos"]])[None]
    ba = jnp.concatenate([p1["b_attn"], p2["b_attn"], p3["b_attn"]])[None]
    td6, mt = _edge_mm(rel, wp, bp, wa3, ba)

    cshift = jnp.maximum(0.0, (mt[0, 0] + msrc[0, 0] + mdst[0, 0]) - 40.0)
    cvec = jnp.broadcast_to(cshift, (16,))

    zeros1 = jnp.zeros((NT1, 128), F32)
    houts = []
    for cidx in range(3):
        tf = td6[2 * cidx:2 * cidx + 2].reshape(2 * E1P, 128)
        vf = vbs6[2 * cidx:2 * cidx + 2].reshape(2 * NT1, 128)
        bdf = bd6[2 * cidx:2 * cidx + 2].reshape(2 * NT1, 64)
        of = _agg123(tf, vf, bdf, srci, dsti, cvec, zeros1)
        houts.append(of.reshape(2, NT1, 64))

    nk = params["neck"]
    hb = _neck(houts[0], houts[1], houts[2], nk["W"], nk["b"][None],
               nk["g"][None], nk["be"][None])
    pooled = _pool(hb.reshape(N_AA, 20, 1280), 1280)
    p5tab = _aapos(_pad_cols(pos, 8).reshape(N_AA, 20, 8))

    pooled_p = _pad_rows(pooled, NTAB5)
    w5cat = jnp.concatenate([p5["W_lin"], p5["W_src"], p5["W_dst"]], axis=1)
    a5 = _mm(pooled_p, w5cat, 256)
    vbs5, bd5, msrc5, mdst5 = _node5(a5, p5["W_attn"])

    e5pad = jnp.full((E5P - N_AA_EDGES,), N_AA, jnp.int32)
    s5 = jnp.concatenate([aa_edge_index[0], e5pad])
    d5i = jnp.concatenate([aa_edge_index[1], e5pad])
    rel5 = _rel_gather(_pad_rows(p5tab, NTAB5), s5, d5i, E5P, 64)

    wp5 = jnp.zeros((16, 1280), F32)
    wp5 = wp5.at[0:3].set(p5["W_pos"][0:3]).at[8:11].set(p5["W_pos"][3:6])
    t5, d5e, mt5 = _edge5(rel5, wp5, p5["b_pos"][None], p5["W_attn"],
                          p5["b_attn"][None])
    cshift5 = jnp.maximum(
        0.0, (mt5[0, 0] + msrc5[0, 0] + mdst5[0, 0]) - 40.0)
    cvec5 = jnp.broadcast_to(cshift5, (16,))

    zeros5 = jnp.zeros((NTAB5, 640), F32)
    o5 = _agg5(t5.reshape(2 * E5P, 640), d5e.reshape(2 * E5P, 640),
               vbs5.reshape(2 * NTAB5, 1280), bd5.reshape(2 * NTAB5, 640),
               s5, d5i, cvec5, zeros5)
    h5 = o5.reshape(2, NTAB5, 640)

    e1, e2, e3, e4, po = (params["esm1"], params["esm2"], params["esm3"],
                          params["esm4"], params["only"])
    tw = [
        _pad_cols(e1["W"], 256), _pad_cols(e1["b"][None], 256),
        _pad_cols(e1["g"][None], 256), _pad_cols(e1["be"][None], 256),
        _pad_cols(_pad_rows(e2["W"], 256), 128),
        _pad_cols(e2["b"][None], 128), _pad_cols(e2["g"][None], 128),
        _pad_cols(e2["be"][None], 128),
        _pad_cols(_pad_rows(e3["W"], 128), 128),
        _pad_cols(e3["b"][None], 128), _pad_cols(e3["g"][None], 128),
        _pad_cols(e3["be"][None], 128),
        _pad_cols(_pad_rows(e4["W"], 128), 128),
        _pad_cols(e4["b"][None], 128),
        _pad_cols((po["W"][:, 1] - po["W"][:, 0])[:, None], 128),
        jnp.broadcast_to(po["b"][1] - po["b"][0], (1, 128)),
    ]
    mask20 = _pad_rows(_pad_cols(mask.reshape(N_AA, 20), 32), NTAB5)
    res = _towers(h5, _pad_rows(esm_list, NTAB5), tw, mask20)
    return res[:N_AA, 0:1]


# speedup vs baseline: 1.0549x; 1.0549x over previous
import functools

import jax
import jax.numpy as jnp
from jax import lax
from jax.experimental import pallas as pl
from jax.experimental.pallas import tpu as pltpu
from jax.experimental.pallas import tpu_sc as plsc

N_NODES = 10000
N_EDGES = 320000
N_AA = 500
N_AA_EDGES = 16000

NT1 = 10240
E1P = 327680
NTAB5 = 512
E5P = 16384

F32 = jnp.float32


def _node_proj_kernel(x_ref, wl_ref, ws_ref, wd_ref, wa_ref,
                      vbs_ref, bd_ref, ms_ref, md_ref):
    i = pl.program_id(0)
    xb = x_ref[...]
    v = jnp.dot(xb, wl_ref[...], preferred_element_type=F32)
    asrc = jnp.dot(xb, ws_ref[...], preferred_element_type=F32)
    adst = jnp.dot(xb, wd_ref[...], preferred_element_type=F32)
    bs_parts = []
    bd_parts = []
    for c in range(3):
        wa = wa_ref[c]
        bs_parts.append(jnp.dot(asrc[:, 128 * c:128 * (c + 1)], wa,
                                preferred_element_type=F32))
        bd_parts.append(jnp.dot(adst[:, 128 * c:128 * (c + 1)], wa,
                                preferred_element_type=F32))
    bs = jnp.concatenate(bs_parts, axis=1)
    bd = jnp.concatenate(bd_parts, axis=1)
    for j in range(6):
        sl = slice(64 * j, 64 * (j + 1))
        vbs_ref[j] = jnp.concatenate([v[:, sl], bs[:, sl]], axis=1)
        bd_ref[j] = bd[:, sl]

    @pl.when(i == 0)
    def _():
        ms_ref[0, 0] = 0.0
        md_ref[0, 0] = 0.0

    ms_ref[0, 0] = jnp.maximum(ms_ref[0, 0], jnp.abs(bs).max())
    md_ref[0, 0] = jnp.maximum(md_ref[0, 0], jnp.abs(bd).max())


def _node_proj(x_p, wl, ws, wd, wa3):
    R = 1024
    return pl.pallas_call(
        _node_proj_kernel,
        grid=(NT1 // R,),
        in_specs=[
            pl.BlockSpec((R, 128), lambda i: (i, 0)),
            pl.BlockSpec((128, 384), lambda i: (0, 0)),
            pl.BlockSpec((128, 384), lambda i: (0, 0)),
            pl.BlockSpec((128, 384), lambda i: (0, 0)),
            pl.BlockSpec((3, 128, 128), lambda i: (0, 0, 0)),
        ],
        out_specs=[
            pl.BlockSpec((6, R, 128), lambda i: (0, i, 0)),
            pl.BlockSpec((6, R, 64), lambda i: (0, i, 0)),
            pl.BlockSpec(memory_space=pltpu.SMEM, block_shape=(1, 1),
                         index_map=lambda i: (0, 0)),
            pl.BlockSpec(memory_space=pltpu.SMEM, block_shape=(1, 1),
                         index_map=lambda i: (0, 0)),
        ],
        out_shape=[
            jax.ShapeDtypeStruct((6, NT1, 128), F32),
            jax.ShapeDtypeStruct((6, NT1, 64), F32),
            jax.ShapeDtypeStruct((1, 1), F32),
            jax.ShapeDtypeStruct((1, 1), F32),
        ],
    )(x_p, wl, ws, wd, wa3)


def _edge_mm_kernel(rel_ref, wp_ref, bp_ref, wa_ref, ba_ref, td_ref, mt_ref):
    i = pl.program_id(0)
    delta = jnp.maximum(
        jnp.dot(rel_ref[...], wp_ref[...], preferred_element_type=F32)
        + bp_ref[...], 0.0)
    t_parts = []
    for c in range(3):
        t_parts.append(jnp.dot(delta[:, 128 * c:128 * (c + 1)], wa_ref[c],
                               preferred_element_type=F32))
    t0 = jnp.concatenate(t_parts, axis=1) + ba_ref[...]
    for j in range(6):
        sl = slice(64 * j, 64 * (j + 1))
        td_ref[j] = jnp.concatenate([t0[:, sl], delta[:, sl]], axis=1)

    @pl.when(i == 0)
    def _():
        mt_ref[0, 0] = 0.0

    mt_ref[0, 0] = jnp.maximum(mt_ref[0, 0], t0.max())


def _edge_mm(rel, wp, bp, wa3, ba):
    EB = 1024
    return pl.pallas_call(
        _edge_mm_kernel,
        grid=(E1P // EB,),
        in_specs=[
            pl.BlockSpec((EB, 16), lambda i: (i, 0)),
            pl.BlockSpec((16, 384), lambda i: (0, 0)),
            pl.BlockSpec((1, 384), lambda i: (0, 0)),
            pl.BlockSpec((3, 128, 128), lambda i: (0, 0, 0)),
            pl.BlockSpec((1, 384), lambda i: (0, 0)),
        ],
        out_specs=[
            pl.BlockSpec((6, EB, 128), lambda i: (0, i, 0)),
            pl.BlockSpec(memory_space=pltpu.SMEM, block_shape=(1, 1),
                         index_map=lambda i: (0, 0)),
        ],
        out_shape=[
            jax.ShapeDtypeStruct((6, E1P, 128), F32),
            jax.ShapeDtypeStruct((1, 1), F32),
        ],
    )(rel, wp, bp, wa3, ba)


def _neck_kernel(h1_ref, h2_ref, h3_ref, w_ref, b_ref, g_ref, be_ref, o_ref):
    h = jnp.concatenate([h1_ref[0], h1_ref[1], h2_ref[0], h2_ref[1],
                         h3_ref[0], h3_ref[1]], axis=1)
    z = jnp.dot(h, w_ref[...], preferred_element_type=F32) + b_ref[...]
    mu = jnp.mean(z, axis=-1, keepdims=True)
    var = jnp.mean((z - mu) * (z - mu), axis=-1, keepdims=True)
    zn = (z - mu) * lax.rsqrt(var + 1e-5) * g_ref[...] + be_ref[...]
    o_ref[...] = jax.nn.gelu(zn)


def _neck(h1, h2, h3, w, b, g, be):
    R = 1000
    return pl.pallas_call(
        _neck_kernel,
        grid=(N_NODES // R,),
        in_specs=[
            pl.BlockSpec((2, R, 64), lambda i: (0, i, 0)),
            pl.BlockSpec((2, R, 64), lambda i: (0, i, 0)),
            pl.BlockSpec((2, R, 64), lambda i: (0, i, 0)),
            pl.BlockSpec((384, 1280), lambda i: (0, 0)),
            pl.BlockSpec((1, 1280), lambda i: (0, 0)),
            pl.BlockSpec((1, 1280), lambda i: (0, 0)),
            pl.BlockSpec((1, 1280), lambda i: (0, 0)),
        ],
        out_specs=pl.BlockSpec((R, 1280), lambda i: (i, 0)),
        out_shape=jax.ShapeDtypeStruct((N_NODES, 1280), F32),
    )(h1, h2, h3, w, b, g, be)


def _pool_kernel(h_ref, o_ref):
    o_ref[...] = jnp.mean(h_ref[...], axis=1)


def _pool(h_r, cols):
    CB = min(cols, 256)
    return pl.pallas_call(
        _pool_kernel,
        grid=(cols // CB,),
        in_specs=[pl.BlockSpec((N_AA, 20, CB), lambda i: (0, 0, i))],
        out_specs=pl.BlockSpec((N_AA, CB), lambda i: (0, i)),
        out_shape=jax.ShapeDtypeStruct((N_AA, cols), F32),
    )(h_r)


def _aapos_kernel(p_ref, o_ref):
    ap = jnp.mean(p_ref[...], axis=1)
    c = ap - jnp.mean(ap, axis=0, keepdims=True)
    nrm = jnp.sqrt(jnp.sum(c * c, axis=1, keepdims=True))
    an = c / (nrm + 1e-8)
    o_ref[...] = jnp.concatenate([ap, an], axis=1)


def _aapos(pos_r):
    return pl.pallas_call(
        _aapos_kernel,
        grid=(1,),
        in_specs=[pl.BlockSpec((N_AA, 20, 8), lambda i: (0, 0, 0))],
        out_specs=pl.BlockSpec((N_AA, 16), lambda i: (0, 0)),
        out_shape=jax.ShapeDtypeStruct((N_AA, 16), F32),
    )(pos_r)


def _mm_kernel(x_ref, w_ref, o_ref):
    o_ref[...] = jnp.dot(x_ref[...], w_ref[...], preferred_element_type=F32)


def _mm(x, w, nb):
    M, K = x.shape
    N = w.shape[1]
    return pl.pallas_call(
        _mm_kernel,
        grid=(N // nb,),
        in_specs=[
            pl.BlockSpec((M, K), lambda i: (0, 0)),
            pl.BlockSpec((K, nb), lambda i: (0, i)),
        ],
        out_specs=pl.BlockSpec((M, nb), lambda i: (0, i)),
        out_shape=jax.ShapeDtypeStruct((M, N), F32),
    )(x, w)


def _node5_kernel(al_ref, as_ref, ad_ref, wa_ref, vbs_ref, bd_ref,
                  ms_ref, md_ref):
    n = pl.program_id(0)
    wa = wa_ref[...]
    bs = jnp.dot(as_ref[...], wa, preferred_element_type=F32)
    bd = jnp.dot(ad_ref[...], wa, preferred_element_type=F32)
    vbs_ref[0] = jnp.concatenate([al_ref[...], bs], axis=1)
    bd_ref[0] = bd

    @pl.when(n == 0)
    def _():
        ms_ref[0, 0] = 0.0
        md_ref[0, 0] = 0.0

    ms_ref[0, 0] = jnp.maximum(ms_ref[0, 0], jnp.abs(bs).max())
    md_ref[0, 0] = jnp.maximum(md_ref[0, 0], jnp.abs(bd).max())


def _node5(a5, wa5):
    return pl.pallas_call(
        _node5_kernel,
        grid=(10,),
        in_specs=[
            pl.BlockSpec((NTAB5, 128), lambda n: (0, n)),
            pl.BlockSpec((NTAB5, 1280), lambda n: (0, 1)),
            pl.BlockSpec((NTAB5, 1280), lambda n: (0, 2)),
            pl.BlockSpec((1280, 128), lambda n: (0, n)),
        ],
        out_specs=[
            pl.BlockSpec((1, NTAB5, 256), lambda n: (n // 5, 0, n % 5)),
            pl.BlockSpec((1, NTAB5, 128), lambda n: (n // 5, 0, n % 5)),
            pl.BlockSpec(memory_space=pltpu.SMEM, block_shape=(1, 1),
                         index_map=lambda n: (0, 0)),
            pl.BlockSpec(memory_space=pltpu.SMEM, block_shape=(1, 1),
                         index_map=lambda n: (0, 0)),
        ],
        out_shape=[
            jax.ShapeDtypeStruct((2, NTAB5, 1280), F32),
            jax.ShapeDtypeStruct((2, NTAB5, 640), F32),
            jax.ShapeDtypeStruct((1, 1), F32),
            jax.ShapeDtypeStruct((1, 1), F32),
        ],
    )(a5, a5, a5, wa5)


def _edge5_kernel(rel_ref, wp_ref, bp_ref, wa_ref, ba_ref, t_ref, d_ref,
                  mt_ref):
    e = pl.program_id(0)
    n = pl.program_id(1)
    delta = jnp.maximum(
        jnp.dot(rel_ref[...], wp_ref[...], preferred_element_type=F32)
        + bp_ref[...], 0.0)
    t0 = jnp.dot(delta, wa_ref[...], preferred_element_type=F32) + ba_ref[...]
    t_ref[0] = t0

    @pl.when(n % 5 == 0)
    def _():
        d_ref[0] = jnp.where(n < 5, delta[:, :640], delta[:, 640:])

    @pl.when(jnp.logical_and(e == 0, n == 0))
    def _():
        mt_ref[0, 0] = 0.0

    mt_ref[0, 0] = jnp.maximum(mt_ref[0, 0], t0.max())


def _edge5(rel5, wp, bp, wa, ba):
    EB = 800
    return pl.pallas_call(
        _edge5_kernel,
        grid=(E5P // EB, 10),
        in_specs=[
            pl.BlockSpec((EB, 16), lambda e, n: (e, 0)),
            pl.BlockSpec((16, 1280), lambda e, n: (0, 0)),
            pl.BlockSpec((1, 1280), lambda e, n: (0, 0)),
            pl.BlockSpec((1280, 128), lambda e, n: (0, n)),
            pl.BlockSpec((1, 128), lambda e, n: (0, n)),
        ],
        out_specs=[
            pl.BlockSpec((1, EB, 128), lambda e, n: (n // 5, e, n % 5)),
            pl.BlockSpec((1, EB, 640), lambda e, n: (n // 5, e, 0)),
            pl.BlockSpec(memory_space=pltpu.SMEM, block_shape=(1, 1),
                         index_map=lambda e, n: (0, 0)),
        ],
        out_shape=[
            jax.ShapeDtypeStruct((2, E5P, 640), F32),
            jax.ShapeDtypeStruct((2, E5P, 640), F32),
            jax.ShapeDtypeStruct((1, 1), F32),
        ],
    )(rel5, wp, bp, wa, ba)


def _towers_kernel(h_ref, esm_ref, w1_ref, b1_ref, g1_ref, e1_ref,
                   w2_ref, b2_ref, g2_ref, e2_ref,
                   w3_ref, b3_ref, g3_ref, e3_ref,
                   w4_ref, b4_ref, wo_ref, bo_ref, mk_ref, o_ref):
    def ln_gelu(z, nch, g, be):
        colmask = (lax.broadcasted_iota(jnp.int32, z.shape, 1) < nch)
        zm = jnp.where(colmask, z, 0.0)
        mu = jnp.sum(zm, axis=-1, keepdims=True) / nch
        dv = jnp.where(colmask, z - mu, 0.0)
        var = jnp.sum(dv * dv, axis=-1, keepdims=True) / nch
        zn = (z - mu) * lax.rsqrt(var + 1e-5) * g + be
        return jax.nn.gelu(jnp.where(colmask, zn, 0.0))

    h2 = jnp.concatenate([h_ref[0], h_ref[1]], axis=1) + esm_ref[...]
    t1 = ln_gelu(jnp.dot(h2, w1_ref[...], preferred_element_type=F32)
                 + b1_ref[...], 150, g1_ref[...], e1_ref[...])
    t2 = ln_gelu(jnp.dot(t1, w2_ref[...], preferred_element_type=F32)
                 + b2_ref[...], 120, g2_ref[...], e2_ref[...])
    t3 = ln_gelu(jnp.dot(t2, w3_ref[...], preferred_element_type=F32)
                 + b3_ref[...], 45, g3_ref[...], e3_ref[...])
    t4 = jnp.dot(t3, w4_ref[...], preferred_element_type=F32) + b4_ref[...]
    z = (jnp.dot(esm_ref[...], wo_ref[...], preferred_element_type=F32)
         + bo_ref[...])
    col = 1.0 / (1.0 + jnp.exp(-z))
    mp = jnp.max(mk_ref[...], axis=1, keepdims=True)
    o_ref[...] = (t4 + col) * mp


def _towers(h5, esm, tw, mask20):
    return pl.pallas_call(
        _towers_kernel,
        grid=(1,),
        in_specs=[
            pl.BlockSpec((2, NTAB5, 640), lambda i: (0, 0, 0)),
            pl.BlockSpec((NTAB5, 1280), lambda i: (0, 0)),
            pl.BlockSpec((1280, 256), lambda i: (0, 0)),
            pl.BlockSpec((1, 256), lambda i: (0, 0)),
            pl.BlockSpec((1, 256), lambda i: (0, 0)),
            pl.BlockSpec((1, 256), lambda i: (0, 0)),
            pl.BlockSpec((256, 128), lambda i: (0, 0)),
            pl.BlockSpec((1, 128), lambda i: (0, 0)),
            pl.BlockSpec((1, 128), lambda i: (0, 0)),
            pl.BlockSpec((1, 128), lambda i: (0, 0)),
            pl.BlockSpec((128, 128), lambda i: (0, 0)),
            pl.BlockSpec((1, 128), lambda i: (0, 0)),
            pl.BlockSpec((1, 128), lambda i: (0, 0)),
            pl.BlockSpec((1, 128), lambda i: (0, 0)),
            pl.BlockSpec((128, 128), lambda i: (0, 0)),
            pl.BlockSpec((1, 128), lambda i: (0, 0)),
            pl.BlockSpec((1280, 128), lambda i: (0, 0)),
            pl.BlockSpec((1, 128), lambda i: (0, 0)),
            pl.BlockSpec((NTAB5, 32), lambda i: (0, 0)),
        ],
        out_specs=pl.BlockSpec((NTAB5, 128), lambda i: (0, 0)),
        out_shape=jax.ShapeDtypeStruct((NTAB5, 128), F32),
    )(h5, esm, *tw, mask20)


def _sc_mesh():
    return plsc.VectorSubcoreMesh(core_axis_name="c", subcore_axis_name="s")


_SC_PARAMS = pltpu.CompilerParams(use_tc_tiling_on_sc=False)


def _rel_gather(ptab, srci, dsti, n_edges, block):
    ew = n_edges // 32
    iters = ew // block

    @functools.partial(
        pl.kernel,
        out_type=jax.ShapeDtypeStruct((n_edges, 16), F32),
        mesh=_sc_mesh(),
        compiler_params=_SC_PARAMS,
        scratch_types=[
            pltpu.VMEM((block,), jnp.int32),
            pltpu.VMEM((block,), jnp.int32),
            pltpu.VMEM((block, 16), F32),
            pltpu.VMEM((block, 16), F32),
            pltpu.VMEM((block, 16), F32),
            pltpu.SemaphoreType.DMA,
        ],
    )
    def k(p_hbm, s_hbm, d_hbm, rel_hbm, si_v, di_v, ps_v, pd_v, rl_v, sem):
        c = lax.axis_index("c")
        s = lax.axis_index("s")
        wid = s * 2 + c
        base = wid * ew

        def body(j, carry):
            b0 = base + j * block
            pltpu.sync_copy(s_hbm.at[pl.ds(b0, block)], si_v)
            pltpu.sync_copy(d_hbm.at[pl.ds(b0, block)], di_v)
            cp1 = pltpu.async_copy(p_hbm.at[si_v], ps_v, sem)
            cp2 = pltpu.async_copy(p_hbm.at[di_v], pd_v, sem)
            cp1.wait()
            cp2.wait()

            @plsc.parallel_loop(0, block, 1, unroll=8)
            def row(r):
                rl_v[r] = pd_v[r] - ps_v[r]
            pltpu.sync_copy(rl_v, rel_hbm.at[pl.ds(b0, block)])
            return carry

        lax.fori_loop(0, iters, body, 0)

    return k(ptab, srci, dsti)


def _agg123(td, vbs, bd, srci, dsti, cvec, zeros):
    n_edges, n_tab, n_acc, dh, B = E1P, NT1, NT1, 64, 64
    et = n_edges // 16
    iters = et // B
    n2 = iters // 2
    rows_pt = n_acc // 16
    rchunk = 32
    riters = rows_pt // rchunk

    @functools.partial(
        pl.kernel,
        out_type=jax.ShapeDtypeStruct((2 * n_acc, dh), F32),
        mesh=_sc_mesh(),
        compiler_params=_SC_PARAMS,
        scratch_types=[
            [pltpu.VMEM((B,), jnp.int32)] * 2,
            [pltpu.VMEM((B,), jnp.int32)] * 2,
            [pltpu.VMEM((B,), jnp.int32)] * 2,
            [pltpu.VMEM((B,), jnp.int32)] * 2,
            [pltpu.VMEM((B, 2 * dh), F32)] * 2,
            [pltpu.VMEM((B, 2 * dh), F32)] * 2,
            [pltpu.VMEM((B, dh), F32)] * 2,
            pltpu.VMEM((16,), F32),
            pltpu.VMEM((rchunk, 2 * dh), F32),
            pltpu.VMEM((rchunk, dh), F32),
            pltpu.VMEM_SHARED((n_acc, 2 * dh), F32),
            [pltpu.SemaphoreType.DMA] * 2,
        ],
    )
    def k(td_hbm, vbs_hbm, bd_hbm, s_hbm, dd_hbm, c_hbm, z_hbm, o_hbm,
          si_v, di_v, gs_v, gd_v, td_v, vbs_v, bd_v, c_v, rd_v, ob_v,
          acc, sems):
        c = lax.axis_index("c")
        s = lax.axis_index("s")
        pltpu.sync_copy(z_hbm.at[pl.ds(s * rows_pt, rows_pt)],
                        acc.at[pl.ds(s * rows_pt, rows_pt)])
        pltpu.sync_copy(c_hbm, c_v)
        plsc.subcore_barrier()
        cval = c_v[...]
        ebase = s * et
        toff = c * n_edges
        noff = c * n_tab

        def issue(sl, b0):
            pltpu.sync_copy(s_hbm.at[pl.ds(b0, B)], si_v[sl])
            pltpu.sync_copy(dd_hbm.at[pl.ds(b0, B)], di_v[sl])

            @plsc.parallel_loop(0, B // 16, 1, unroll=4)
            def oset(r):
                gs_v[sl][pl.ds(r * 16, 16)] = (
                    si_v[sl][pl.ds(r * 16, 16)] + noff)
                gd_v[sl][pl.ds(r * 16, 16)] = (
                    di_v[sl][pl.ds(r * 16, 16)] + noff)
            pltpu.async_copy(td_hbm.at[pl.ds(toff + b0, B)], td_v[sl],
                             sems[sl])
            pltpu.async_copy(vbs_hbm.at[gs_v[sl]], vbs_v[sl], sems[sl])
            pltpu.async_copy(bd_hbm.at[gd_v[sl]], bd_v[sl], sems[sl])

        def wait(sl, b0):
            pltpu.make_async_copy(td_hbm.at[pl.ds(toff + b0, B)], td_v[sl],
                                  sems[sl]).wait()
            pltpu.make_async_copy(vbs_hbm.at[gs_v[sl]], vbs_v[sl],
                                  sems[sl]).wait()
            pltpu.make_async_copy(bd_hbm.at[gd_v[sl]], bd_v[sl],
                                  sems[sl]).wait()

        def compute_scatter(sl):
            tdb = td_v[sl]
            vbsb = vbs_v[sl]
            bdb = bd_v[sl]

            @plsc.parallel_loop(0, B, 1, unroll=8)
            def rows(r):
                for kk in range(4):
                    a = pl.ds(16 * kk, 16)
                    b = pl.ds(dh + 16 * kk, 16)
                    t0 = tdb[r, a]
                    d = tdb[r, b]
                    al = jnp.maximum(t0 + bdb[r, a] - vbsb[r, b], 0.0)
                    e = jnp.exp(al - cval)
                    tdb[r, a] = e
                    tdb[r, b] = e * (vbsb[r, a] + d)

            pltpu.sync_copy(tdb, acc.at[di_v[sl]], add=True)

        issue(0, ebase)

        def body(j2, carry):
            jA = ebase + (2 * j2) * B
            jB = jA + B
            issue(1, jB)
            wait(0, jA)
            compute_scatter(0)

            @pl.when(j2 < n2 - 1)
            def _():
                issue(0, jB + B)

            wait(1, jB)
            compute_scatter(1)
            return carry

        lax.fori_loop(0, n2, body, 0)
        plsc.subcore_barrier()

        def rbody(j, carry):
            r0 = s * rows_pt + j * rchunk
            pltpu.sync_copy(acc.at[pl.ds(r0, rchunk)], rd_v)

            @plsc.parallel_loop(0, rchunk, 1, unroll=8)
            def rrow(r):
                for kk in range(4):
                    ob_v[r, pl.ds(16 * kk, 16)] = (
                        rd_v[r, pl.ds(dh + 16 * kk, 16)]
                        / (rd_v[r, pl.ds(16 * kk, 16)] + 1e-30))
            pltpu.sync_copy(ob_v, o_hbm.at[pl.ds(c * n_acc + r0, rchunk)])
            return carry

        lax.fori_loop(0, riters, rbody, 0)

    return k(td, vbs, bd, srci, dsti, cvec, zeros)


def _agg5(t5, d5, vbs5, bd5, srci, dsti, cvec, zeros):
    n_edges, n_tab, n_acc, dh, B = E5P, NTAB5, NTAB5, 640, 16
    et = n_edges // 16
    iters = et // B
    rows_pt = n_acc // 16
    rchunk = 8
    riters = rows_pt // rchunk

    @functools.partial(
        pl.kernel,
        out_type=jax.ShapeDtypeStruct((2 * n_acc, dh), F32),
        mesh=_sc_mesh(),
        compiler_params=_SC_PARAMS,
        scratch_types=[
            pltpu.VMEM((B,), jnp.int32),
            pltpu.VMEM((B,), jnp.int32),
            pltpu.VMEM((B,), jnp.int32),
            pltpu.VMEM((B,), jnp.int32),
            pltpu.VMEM((B, dh), F32),
            pltpu.VMEM((B, dh), F32),
            pltpu.VMEM((B, 2 * dh), F32),
            pltpu.VMEM((B, dh), F32),
            pltpu.VMEM((16,), F32),
            pltpu.VMEM((rchunk, dh), F32),
            pltpu.VMEM((rchunk, dh), F32),
            pltpu.VMEM_SHARED((n_acc, dh), F32),
            pltpu.VMEM_SHARED((n_acc, dh), F32),
            pltpu.SemaphoreType.DMA,
        ],
    )
    def k(t_hbm, d_hbm, vbs_hbm, bd_hbm, s_hbm, dd_hbm, c_hbm, z_hbm, o_hbm,
          si_v, di_v, gs_v, gd_v, t_v, d_v, vbs_v, bd_v, c_v, rn_v, rdn_v,
          accn, accd, sem):
        c = lax.axis_index("c")
        s = lax.axis_index("s")
        pltpu.sync_copy(z_hbm.at[pl.ds(s * rows_pt, rows_pt)],
                        accn.at[pl.ds(s * rows_pt, rows_pt)])
        pltpu.sync_copy(z_hbm.at[pl.ds(s * rows_pt, rows_pt)],
                        accd.at[pl.ds(s * rows_pt, rows_pt)])
        pltpu.sync_copy(c_hbm, c_v)
        plsc.subcore_barrier()
        cval = c_v[...]
        ebase = s * et
        toff = c * n_edges
        noff = c * n_tab

        def body(j, carry):
            b0 = ebase + j * B
            pltpu.sync_copy(s_hbm.at[pl.ds(b0, B)], si_v)
            pltpu.sync_copy(dd_hbm.at[pl.ds(b0, B)], di_v)
            gs_v[...] = si_v[...] + noff
            gd_v[...] = di_v[...] + noff
            cp1 = pltpu.async_copy(t_hbm.at[pl.ds(toff + b0, B)], t_v, sem)
            cp2 = pltpu.async_copy(d_hbm.at[pl.ds(toff + b0, B)], d_v, sem)
            cp3 = pltpu.async_copy(vbs_hbm.at[gs_v], vbs_v, sem)
            cp4 = pltpu.async_copy(bd_hbm.at[gd_v], bd_v, sem)
            cp1.wait()
            cp2.wait()
            cp3.wait()
            cp4.wait()

            @plsc.parallel_loop(0, B, 1, unroll=2)
            def rows(r):
                for kk in range(40):
                    vcol = 256 * (kk // 8) + 16 * (kk % 8)
                    a = pl.ds(16 * kk, 16)
                    t0 = t_v[r, a]
                    d = d_v[r, a]
                    al = jnp.maximum(
                        t0 + bd_v[r, a] - vbs_v[r, pl.ds(vcol + 128, 16)],
                        0.0)
                    e = jnp.exp(al - cval)
                    t_v[r, a] = e
                    d_v[r, a] = e * (vbs_v[r, pl.ds(vcol, 16)] + d)
            pltpu.sync_copy(d_v, accn.at[di_v], add=True)
            pltpu.sync_copy(t_v, accd.at[di_v], add=True)
            return carry

        lax.fori_loop(0, iters, body, 0)
        plsc.subcore_barrier()

        def rbody(j, carry):
            r0 = s * rows_pt + j * rchunk
            pltpu.sync_copy(accn.at[pl.ds(r0, rchunk)], rn_v)
            pltpu.sync_copy(accd.at[pl.ds(r0, rchunk)], rdn_v)

            @plsc.parallel_loop(0, rchunk, 1, unroll=2)
            def rrow(r):
                for kk in range(40):
                    a = pl.ds(16 * kk, 16)
                    rn_v[r, a] = rn_v[r, a] / (rdn_v[r, a] + 1e-30)
            pltpu.sync_copy(rn_v, o_hbm.at[pl.ds(c * n_acc + r0, rchunk)])
            return carry

        lax.fori_loop(0, riters, rbody, 0)

    return k(t5, d5, vbs5, bd5, srci, dsti, cvec, zeros)


def _pad_cols(a, n):
    return jnp.pad(a, ((0, 0), (0, n - a.shape[1])))


def _pad_rows(a, n):
    return jnp.pad(a, ((0, n - a.shape[0]), (0, 0)))


def kernel(x, pos, normal, mask, esm_list, edge_index, aa_edge_index,
           pool_batch, params):
    p1, p2, p3, p5 = (params["conv1"], params["conv2"], params["conv3"],
                      params["conv5"])
    epad = jnp.full((E1P - N_EDGES,), N_NODES, jnp.int32)
    srci = jnp.concatenate([edge_index[0], epad])
    dsti = jnp.concatenate([edge_index[1], epad])

    x_p = _pad_rows(_pad_cols(x, 128), NT1)
    wl = _pad_rows(jnp.concatenate([p1["W_lin"], p2["W_lin"], p3["W_lin"]],
                                   axis=1), 128)
    ws = _pad_rows(jnp.concatenate([p1["W_src"], p2["W_src"], p3["W_src"]],
                                   axis=1), 128)
    wd = _pad_rows(jnp.concatenate([p1["W_dst"], p2["W_dst"], p3["W_dst"]],
                                   axis=1), 128)
    wa3 = jnp.stack([p1["W_attn"], p2["W_attn"], p3["W_attn"]])
    vbs6, bd6, msrc, mdst = _node_proj(x_p, wl, ws, wd, wa3)

    ptab = _pad_rows(jnp.concatenate(
        [_pad_cols(pos, 8), _pad_cols(normal, 8)], axis=1), NT1)
    rel = _rel_gather(ptab, srci, dsti, E1P, 128)

    wp = jnp.zeros((16, 384), F32)
    wp_all = jnp.concatenate([p1["W_pos"], p2["W_pos"], p3["W_pos"]], axis=1)
    wp = wp.at[0:3].set(wp_all[0:3]).at[8:11].set(wp_all[3:6])
    bp = jnp.concatenate([p1["b_pos"], p2["b_pos"], p3["b_pos"]])[None]
    ba = jnp.concatenate([p1["b_attn"], p2["b_attn"], p3["b_attn"]])[None]
    td6, mt = _edge_mm(rel, wp, bp, wa3, ba)

    cshift = jnp.maximum(0.0, (mt[0, 0] + msrc[0, 0] + mdst[0, 0]) - 40.0)
    cvec = jnp.broadcast_to(cshift, (16,))

    zeros1 = jnp.zeros((NT1, 128), F32)
    houts = []
    for cidx in range(3):
        tf = td6[2 * cidx:2 * cidx + 2].reshape(2 * E1P, 128)
        vf = vbs6[2 * cidx:2 * cidx + 2].reshape(2 * NT1, 128)
        bdf = bd6[2 * cidx:2 * cidx + 2].reshape(2 * NT1, 64)
        of = _agg123(tf, vf, bdf, srci, dsti, cvec, zeros1)
        houts.append(of.reshape(2, NT1, 64))

    nk = params["neck"]
    hb = _neck(houts[0], houts[1], houts[2], nk["W"], nk["b"][None],
               nk["g"][None], nk["be"][None])
    pooled = _pool(hb.reshape(N_AA, 20, 1280), 1280)
    p5tab = _aapos(_pad_cols(pos, 8).reshape(N_AA, 20, 8))

    pooled_p = _pad_rows(pooled, NTAB5)
    w5cat = jnp.concatenate([p5["W_lin"], p5["W_src"], p5["W_dst"]], axis=1)
    a5 = _mm(pooled_p, w5cat, 256)
    vbs5, bd5, msrc5, mdst5 = _node5(a5, p5["W_attn"])

    e5pad = jnp.full((E5P - N_AA_EDGES,), N_AA, jnp.int32)
    s5 = jnp.concatenate([aa_edge_index[0], e5pad])
    d5i = jnp.concatenate([aa_edge_index[1], e5pad])
    rel5 = _rel_gather(_pad_rows(p5tab, NTAB5), s5, d5i, E5P, 64)

    wp5 = jnp.zeros((16, 1280), F32)
    wp5 = wp5.at[0:3].set(p5["W_pos"][0:3]).at[8:11].set(p5["W_pos"][3:6])
    t5, d5e, mt5 = _edge5(rel5, wp5, p5["b_pos"][None], p5["W_attn"],
                          p5["b_attn"][None])
    cshift5 = jnp.maximum(
        0.0, (mt5[0, 0] + msrc5[0, 0] + mdst5[0, 0]) - 40.0)
    cvec5 = jnp.broadcast_to(cshift5, (16,))

    zeros5 = jnp.zeros((NTAB5, 640), F32)
    o5 = _agg5(t5.reshape(2 * E5P, 640), d5e.reshape(2 * E5P, 640),
               vbs5.reshape(2 * NTAB5, 1280), bd5.reshape(2 * NTAB5, 640),
               s5, d5i, cvec5, zeros5)
    h5 = o5.reshape(2, NTAB5, 640)

    e1, e2, e3, e4, po = (params["esm1"], params["esm2"], params["esm3"],
                          params["esm4"], params["only"])
    tw = [
        _pad_cols(e1["W"], 256), _pad_cols(e1["b"][None], 256),
        _pad_cols(e1["g"][None], 256), _pad_cols(e1["be"][None], 256),
        _pad_cols(_pad_rows(e2["W"], 256), 128),
        _pad_cols(e2["b"][None], 128), _pad_cols(e2["g"][None], 128),
        _pad_cols(e2["be"][None], 128),
        _pad_cols(_pad_rows(e3["W"], 128), 128),
        _pad_cols(e3["b"][None], 128), _pad_cols(e3["g"][None], 128),
        _pad_cols(e3["be"][None], 128),
        _pad_cols(_pad_rows(e4["W"], 128), 128),
        _pad_cols(e4["b"][None], 128),
        _pad_cols((po["W"][:, 1] - po["W"][:, 0])[:, None], 128),
        jnp.broadcast_to(po["b"][1] - po["b"][0], (1, 128)),
    ]
    mask20 = _pad_rows(_pad_cols(mask.reshape(N_AA, 20), 32), NTAB5)
    res = _towers(h5, _pad_rows(esm_list, NTAB5), tw, mask20)
    return res[:N_AA, 0:1]
